# Initial kernel scaffold; baseline (speedup 1.0000x reference)
import jax
import jax.numpy as jnp
from jax.experimental import pallas as pl


def kernel(x, edge_index, batch, W1, b1, W2, b2):
    raise NotImplementedError("write your pallas kernel here")


# speedup vs baseline: 88.1168x; 88.1168x over previous
import functools

import jax
import jax.numpy as jnp
from jax import lax
from jax.experimental import pallas as pl
from jax.experimental.pallas import tpu as pltpu
from jax.experimental.pallas import tpu_sc as plsc

NC = 2
NS = 16
L = 16
NW = NC * NS
ROW = 128
ALIGN = NW * 8 * ROW

F32 = jnp.float32
I32 = jnp.int32


def _mesh():
    return plsc.VectorSubcoreMesh(core_axis_name="c", subcore_axis_name="s",
                                  num_cores=NC, num_subcores=NS)


_SC_PARAMS = pltpu.CompilerParams(needs_layout_passes=False)


def _zero_fill(buf, n):
    zv = jnp.zeros((L,), F32)

    def body(i, _):
        buf[pl.ds(i * L, L)] = zv
        return 0

    lax.fori_loop(0, n // L, body, 0)


def _tc_matvec(x, W1, W2):
    n, d_in = x.shape
    blk = 400
    grid = n // blk

    def body(x_ref, w1_ref, w2_ref, y_ref):
        w = w1_ref[...] @ w2_ref[...]
        y_ref[...] = x_ref[...] @ w

    return pl.pallas_call(
        body,
        grid=(grid,),
        in_specs=[
            pl.BlockSpec((blk, d_in), lambda i: (i, 0)),
            pl.BlockSpec(W1.shape, lambda i: (0, 0)),
            pl.BlockSpec(W2.shape, lambda i: (0, 0)),
        ],
        out_specs=pl.BlockSpec((blk, 1), lambda i: (i, 0)),
        out_shape=jax.ShapeDtypeStruct((n, 1), F32),
    )(x, W1, W2)


def _sc_deg_cnt(dst2d, batch2d, n_pad, g_pad):
    er_w = dst2d.shape[0] // NW
    br_w = batch2d.shape[0] // NW
    n_sl = n_pad // NS
    g_sl = g_pad // NS

    @functools.partial(
        pl.kernel,
        out_type=(jax.ShapeDtypeStruct((NC * n_pad,), F32),
                  jax.ShapeDtypeStruct((NC * g_pad,), F32)),
        mesh=_mesh(),
        compiler_params=_SC_PARAMS,
        scratch_types=[
            pltpu.VMEM((er_w, ROW), I32),
            pltpu.VMEM((br_w, ROW), I32),
            pltpu.VMEM((ROW,), F32),
            pltpu.VMEM((n_sl,), F32),
            pltpu.VMEM_SHARED((n_pad,), F32),
            pltpu.VMEM_SHARED((g_pad,), F32),
        ],
    )
    def k(dst_hbm, bat_hbm, deg_out, cnt_out, dst_v, bat_v, ones_v, zrb_v,
          acc_n, acc_g):
        cid = lax.axis_index("c")
        sid = lax.axis_index("s")
        wid = cid * NS + sid
        _zero_fill(zrb_v, n_sl)
        for i in range(ROW // L):
            ones_v[pl.ds(i * L, L)] = jnp.ones((L,), F32)
        pltpu.sync_copy(zrb_v, acc_n.at[pl.ds(sid * n_sl, n_sl)])
        pltpu.sync_copy(zrb_v.at[pl.ds(0, g_sl)],
                        acc_g.at[pl.ds(sid * g_sl, g_sl)])
        plsc.subcore_barrier()
        pltpu.sync_copy(dst_hbm.at[pl.ds(wid * er_w, er_w), :], dst_v)
        pltpu.sync_copy(bat_hbm.at[pl.ds(wid * br_w, br_w), :], bat_v)

        def erow(r, _):
            pltpu.sync_copy(ones_v, acc_n.at[dst_v.at[r]], add=True)
            return 0

        lax.fori_loop(0, er_w, erow, 0)

        def brow(r, _):
            pltpu.sync_copy(ones_v, acc_g.at[bat_v.at[r]], add=True)
            return 0

        lax.fori_loop(0, br_w, brow, 0)
        plsc.subcore_barrier()
        pltpu.sync_copy(acc_n.at[pl.ds(sid * n_sl, n_sl)], zrb_v)
        pltpu.sync_copy(zrb_v, deg_out.at[pl.ds(cid * n_pad + sid * n_sl,
                                                n_sl)])
        pltpu.sync_copy(acc_g.at[pl.ds(sid * g_sl, g_sl)],
                        zrb_v.at[pl.ds(0, g_sl)])
        pltpu.sync_copy(zrb_v.at[pl.ds(0, g_sl)],
                        cnt_out.at[pl.ds(cid * g_pad + sid * g_sl, g_sl)])

    return k(dst2d, batch2d)


def _tc_dis_u0(deg_p, y_pad):
    n_pad = y_pad.shape[0]
    r = n_pad // ROW
    deg3 = deg_p.reshape(NC, r, ROW)
    y2 = y_pad.reshape(r, ROW)

    def body(deg_ref, y_ref, dis_ref, u0_ref):
        d = deg_ref[0] + deg_ref[1] + jnp.float32(1.0)
        dis = lax.rsqrt(d)
        dis_ref[...] = dis
        u0_ref[...] = dis * y_ref[...]

    dis, u0 = pl.pallas_call(
        body,
        out_shape=(jax.ShapeDtypeStruct((r, ROW), F32),
                   jax.ShapeDtypeStruct((r, ROW), F32)),
    )(deg3, y2)
    return dis.reshape(n_pad), u0.reshape(n_pad)


def _sc_hop(src2d, dst2d, u):
    n_pad = u.shape[0]
    er_w = src2d.shape[0] // NW
    rb = next(c for c in (64, 56, 48, 40, 32, 24, 16, 8) if er_w % c == 0)
    nblk = er_w // rb
    n_sl = n_pad // NS

    @functools.partial(
        pl.kernel,
        out_type=jax.ShapeDtypeStruct((NC * n_pad,), F32),
        mesh=_mesh(),
        compiler_params=_SC_PARAMS,
        scratch_types=[
            pltpu.VMEM((n_pad,), F32),
            pltpu.VMEM((rb, ROW), I32),
            pltpu.VMEM((rb, ROW), I32),
            pltpu.VMEM((rb, ROW), F32),
            pltpu.VMEM((n_sl,), F32),
            pltpu.VMEM_SHARED((n_pad,), F32),
        ],
    )
    def k(src_hbm, dst_hbm, u_hbm, t_out, u_v, src_v, dst_v, msg_v, zrb_v,
          acc):
        cid = lax.axis_index("c")
        sid = lax.axis_index("s")
        wid = cid * NS + sid
        _zero_fill(zrb_v, n_sl)
        pltpu.sync_copy(zrb_v, acc.at[pl.ds(sid * n_sl, n_sl)])
        pltpu.sync_copy(u_hbm, u_v)
        plsc.subcore_barrier()
        for b in range(nblk):
            row0 = wid * er_w + b * rb
            pltpu.sync_copy(src_hbm.at[pl.ds(row0, rb), :], src_v)
            pltpu.sync_copy(dst_hbm.at[pl.ds(row0, rb), :], dst_v)

            def row(r, _):
                for kk in range(ROW // L):
                    idx = src_v[r, pl.ds(kk * L, L)]
                    msg_v[r, pl.ds(kk * L, L)] = plsc.load_gather(u_v, [idx])
                pltpu.sync_copy(msg_v.at[r], acc.at[dst_v.at[r]], add=True)
                return 0

            lax.fori_loop(0, rb, row, 0)
        plsc.subcore_barrier()
        pltpu.sync_copy(acc.at[pl.ds(sid * n_sl, n_sl)], zrb_v)
        pltpu.sync_copy(zrb_v, t_out.at[pl.ds(cid * n_pad + sid * n_sl,
                                              n_sl)])

    return k(src2d, dst2d, u)


def _tc_u_next(t_p, u, dis):
    n_pad = u.shape[0]
    r = n_pad // ROW
    t3 = t_p.reshape(NC, r, ROW)

    def body(t_ref, u_ref, dis_ref, un_ref):
        d = dis_ref[...]
        un_ref[...] = d * d * (t_ref[0] + t_ref[1] + u_ref[...])

    un = pl.pallas_call(
        body,
        out_shape=jax.ShapeDtypeStruct((r, ROW), F32),
    )(t3, u.reshape(r, ROW), dis.reshape(r, ROW))
    return un.reshape(n_pad)


def _sc_pool(t_p, u, dis, batch2d, g_pad):
    n_pad = u.shape[0]
    ch = n_pad // NW
    br_w = batch2d.shape[0] // NW
    g_sl = g_pad // NS

    @functools.partial(
        pl.kernel,
        out_type=jax.ShapeDtypeStruct((NC * g_pad,), F32),
        mesh=_mesh(),
        compiler_params=_SC_PARAMS,
        scratch_types=[
            pltpu.VMEM((ch,), F32), pltpu.VMEM((ch,), F32),
            pltpu.VMEM((ch,), F32), pltpu.VMEM((ch,), F32),
            pltpu.VMEM((br_w, ROW), I32),
            pltpu.VMEM((g_pad,), F32),
            pltpu.VMEM((g_sl,), F32), pltpu.VMEM((g_sl,), F32),
            pltpu.VMEM_SHARED((NS * g_pad,), F32),
        ],
    )
    def k(tp_hbm, u_hbm, dis_hbm, bat_hbm, pool_out, t0_v, t1_v, u_v, dis_v,
          bat_v, loc_v, sum_v, tmp_v, stage):
        cid = lax.axis_index("c")
        sid = lax.axis_index("s")
        wid = cid * NS + sid
        base = wid * ch
        _zero_fill(loc_v, g_pad)
        pltpu.sync_copy(tp_hbm.at[pl.ds(base, ch)], t0_v)
        pltpu.sync_copy(tp_hbm.at[pl.ds(n_pad + base, ch)], t1_v)
        pltpu.sync_copy(u_hbm.at[pl.ds(base, ch)], u_v)
        pltpu.sync_copy(dis_hbm.at[pl.ds(base, ch)], dis_v)
        pltpu.sync_copy(bat_hbm.at[pl.ds(wid * br_w, br_w), :], bat_v)

        def zbody(r, _):
            for kk in range(ROW // L):
                s = pl.ds(r * ROW + kk * L, L)
                z = dis_v[s] * (t0_v[s] + t1_v[s] + u_v[s])
                plsc.addupdate_scatter(loc_v, [bat_v[r, pl.ds(kk * L, L)]], z)
            return 0

        lax.fori_loop(0, br_w, zbody, 0)
        pltpu.sync_copy(loc_v, stage.at[pl.ds(sid * g_pad, g_pad)])
        plsc.subcore_barrier()
        _zero_fill(sum_v, g_sl)
        for j in range(NS):
            pltpu.sync_copy(stage.at[pl.ds(j * g_pad + sid * g_sl, g_sl)],
                            tmp_v)
            for i in range(g_sl // L):
                s = pl.ds(i * L, L)
                sum_v[s] = sum_v[s] + tmp_v[s]
        pltpu.sync_copy(sum_v, pool_out.at[pl.ds(cid * g_pad + sid * g_sl,
                                                 g_sl)])

    return k(t_p, u, dis, batch2d)


def _tc_final(pool_p, cnt_p, b1, W2, b2, g_pad):
    gr = g_pad // ROW

    def body(pool_ref, cnt_ref, b1_ref, w2_ref, b2_ref, o_ref):
        c1s = jnp.sum(b1_ref[...] * w2_ref[...])
        pooled = pool_ref[0] + pool_ref[1]
        cnt = cnt_ref[0] + cnt_ref[1]
        o_ref[...] = pooled + cnt * c1s + b2_ref[0, 0]

    out = pl.pallas_call(
        body,
        out_shape=jax.ShapeDtypeStruct((gr, ROW), F32),
    )(pool_p.reshape(NC, gr, ROW), cnt_p.reshape(NC, gr, ROW),
      b1.reshape(1, -1), W2.reshape(1, -1), b2.reshape(1, 1))
    return out.reshape(g_pad)


def kernel(x, edge_index, batch, W1, b1, W2, b2):
    n = x.shape[0]
    e = edge_index.shape[1]
    g = 512
    g_pad = 1024
    n_pad = ((n + 1 + ALIGN - 1) // ALIGN) * ALIGN
    e_pad = ((e + ALIGN - 1) // ALIGN) * ALIGN

    src2d = jnp.concatenate(
        [edge_index[0], jnp.zeros((e_pad - e,), I32)]).reshape(-1, ROW)
    dst2d = jnp.concatenate(
        [edge_index[1], jnp.full((e_pad - e,), n, I32)]).reshape(-1, ROW)
    batch2d = jnp.concatenate(
        [batch, jnp.full((n_pad - n,), g, I32)]).reshape(-1, ROW)

    y = _tc_matvec(x, W1, W2)
    y_pad = jnp.concatenate([y[:, 0], jnp.zeros((n_pad - n,), F32)])

    deg_p, cnt_p = _sc_deg_cnt(dst2d, batch2d, n_pad, g_pad)
    dis, u0 = _tc_dis_u0(deg_p, y_pad)
    t1_p = _sc_hop(src2d, dst2d, u0)
    u1 = _tc_u_next(t1_p, u0, dis)
    t2_p = _sc_hop(src2d, dst2d, u1)
    pool_p = _sc_pool(t2_p, u1, dis, batch2d, g_pad)
    out = _tc_final(pool_p, cnt_p, b1, W2, b2, g_pad)
    return out[:g].reshape(g, 1)
